# baseline (speedup 1.0000x reference)
import functools

import jax
import jax.numpy as jnp
from jax import lax
from jax.experimental import pallas as pl
from jax.experimental.pallas import tpu as pltpu
from jax.experimental.pallas import tpu_sc as plsc

B = 4096
EMB = 128
VOCAB = 1000
HP = 1024
L = 50
EPS = 1e-5

_NC, _NS = 2, 16
_NW = _NC * _NS
_RPW = B // _NW
_CH = 32
_NCH = _RPW // _CH


def _sc_body(seq_hbm, zz_hbm, h_hbm,
             seqb0, seqb1, hb0, hb1, sem0, sem1):
    wid = lax.axis_index("s") * _NC + lax.axis_index("c")
    base = wid * _RPW

    zeros16 = jnp.zeros((16,), jnp.float32)
    ones16 = jnp.ones((16,), jnp.float32)
    tail = lax.iota(jnp.int32, 16) >= jnp.int32(4 * 16 - L)

    cz0 = pltpu.async_copy(zz_hbm, hb0, sem0)
    cz1 = pltpu.async_copy(zz_hbm, hb1, sem1)
    cz0.wait()
    cz1.wait()

    _UNR = 4

    def _rows(seqb, hb):
        def body(i, carry):
            for di in range(_UNR):
                r = i * _UNR + di
                row = jnp.broadcast_to(r, (16,)).astype(jnp.int32)
                for j in range(3):
                    col = seqb[pl.ds(r * L + j * 16, 16)]
                    plsc.addupdate_scatter(hb, [row, col], ones16)
                col = seqb[pl.ds(r * L + (L - 16), 16)]
                plsc.addupdate_scatter(hb, [row, col], ones16, mask=tail)
            return carry
        lax.fori_loop(0, _CH // _UNR, body, 0)

    def _unrows(seqb, hb):
        def body(i, carry):
            for di in range(_UNR):
                r = i * _UNR + di
                row = jnp.broadcast_to(r, (16,)).astype(jnp.int32)
                for j in range(3):
                    col = seqb[pl.ds(r * L + j * 16, 16)]
                    plsc.store_scatter(hb, [row, col], zeros16)
                col = seqb[pl.ds(r * L + (L - 16), 16)]
                plsc.store_scatter(hb, [row, col], zeros16, mask=tail)
            return carry
        lax.fori_loop(0, _CH // _UNR, body, 0)

    bufs = ((seqb0, hb0, sem0), (seqb1, hb1, sem1))
    cps = [None, None]
    for c in range(_NCH):
        seqb, hb, sem = bufs[c % 2]
        if cps[c % 2] is not None:
            cps[c % 2].wait()
            _unrows(seqb, hb)
        r0 = base + c * _CH
        pltpu.sync_copy(seq_hbm.at[pl.ds(r0 * L, _CH * L)], seqb)
        _rows(seqb, hb)
        cps[c % 2] = pltpu.async_copy(hb, h_hbm.at[pl.ds(r0, _CH)], sem)
    cps[0].wait()
    cps[1].wait()


@functools.cache
def _sc_embed_call():
    return pl.kernel(
        _sc_body,
        out_type=jax.ShapeDtypeStruct((B, HP), jnp.float32),
        mesh=plsc.VectorSubcoreMesh(core_axis_name="c", subcore_axis_name="s",
                                    num_cores=_NC, num_subcores=_NS),
        scratch_types=[
            pltpu.VMEM((_CH * L,), jnp.int32),
            pltpu.VMEM((_CH * L,), jnp.int32),
            pltpu.VMEM((_CH, HP), jnp.float32),
            pltpu.VMEM((_CH, HP), jnp.float32),
            pltpu.SemaphoreType.DMA,
            pltpu.SemaphoreType.DMA,
        ],
        compiler_params=pltpu.CompilerParams(needs_layout_passes=False),
    )


def _sc_embed(*args):
    return _sc_embed_call()(*args)

TILE = 2048
GRID = B // TILE
_ARB = pltpu.CompilerParams(dimension_semantics=("arbitrary",))


def _stats_update(so_ref, h32):
    @pl.when(pl.program_id(0) == 0)
    def _():
        so_ref[...] = jnp.zeros_like(so_ref)
    so_ref[...] += jnp.concatenate(
        [jnp.sum(h32, 0, keepdims=True),
         jnp.sum(h32 * h32, 0, keepdims=True)], 0)


def _l1_body(h_ref, pos_ref, side_ref, cw_ref, pw_ref, sw_ref,
             W_ref, b_ref, o_ref, so_ref):
    hb = h_ref[...].astype(jnp.bfloat16)
    s = jnp.dot(hb, cw_ref[...], preferred_element_type=jnp.float32)
    n = jnp.maximum(jnp.float32(L) - h_ref[:, 0:1], 1.0)
    m = s / n
    lanes = lax.broadcasted_iota(jnp.int32, (TILE, EMB), 1)
    oh_p = (pos_ref[...] == lanes).astype(jnp.bfloat16)
    oh_s = (side_ref[...] == lanes).astype(jnp.bfloat16)
    p = jnp.dot(oh_p, pw_ref[...], preferred_element_type=jnp.float32)
    d = jnp.dot(oh_s, sw_ref[...], preferred_element_type=jnp.float32)
    x = jnp.concatenate([s, m, p, d], axis=1)
    y = jnp.dot(x.astype(jnp.bfloat16), W_ref[...],
                preferred_element_type=jnp.float32) + b_ref[...]
    hb1 = jnp.maximum(y, 0.0).astype(jnp.bfloat16)
    o_ref[...] = hb1
    _stats_update(so_ref, hb1.astype(jnp.float32))


def _bn_fold(st_ref, g_ref, be_ref):
    mu = st_ref[0:1, :] * (1.0 / B)
    var = st_ref[1:2, :] * (1.0 / B) - mu * mu
    a = g_ref[...] * lax.rsqrt(var + EPS)
    c = be_ref[...] - mu * a
    return a, c


def _mid_body(h_ref, st_ref, g_ref, be_ref, W_ref, b_ref, o_ref, so_ref):
    a, c = _bn_fold(st_ref, g_ref, be_ref)
    z = (h_ref[...].astype(jnp.float32) * a + c).astype(jnp.bfloat16)
    y = jnp.dot(z, W_ref[...], preferred_element_type=jnp.float32) + b_ref[...]
    hb = jnp.maximum(y, 0.0).astype(jnp.bfloat16)
    o_ref[...] = hb
    _stats_update(so_ref, hb.astype(jnp.float32))


def _fin_body(h_ref, st_ref, g_ref, be_ref, W_ref, b_ref, o_ref):
    a, c = _bn_fold(st_ref, g_ref, be_ref)
    z = (h_ref[...].astype(jnp.float32) * a + c).astype(jnp.bfloat16)
    o_ref[...] = jnp.dot(z, W_ref[...],
                         preferred_element_type=jnp.float32) + b_ref[...]


def _row_spec(n):
    return pl.BlockSpec((TILE, n), lambda i: (i, 0))


def _full_spec(m, n):
    return pl.BlockSpec((m, n), lambda i: (0, 0))


def _layer1(h2d, pos_f, side_f, cw, pw, sw, W, b):
    return pl.pallas_call(
        _l1_body,
        grid=(GRID,),
        in_specs=[_row_spec(HP), pl.BlockSpec((TILE, 1), lambda i: (i, 0)),
                  pl.BlockSpec((TILE, 1), lambda i: (i, 0)),
                  _full_spec(HP, EMB), _full_spec(EMB, EMB),
                  _full_spec(EMB, EMB), _full_spec(4 * EMB, 1024),
                  _full_spec(1, 1024)],
        out_specs=[_row_spec(1024), _full_spec(2, 1024)],
        out_shape=[jax.ShapeDtypeStruct((B, 1024), jnp.bfloat16),
                   jax.ShapeDtypeStruct((2, 1024), jnp.float32)],
        compiler_params=_ARB,
    )(h2d, pos_f, side_f, cw, pw, sw, W, b)


def _mid(h, st, g, be, W, b, din, dout):
    return pl.pallas_call(
        _mid_body,
        grid=(GRID,),
        in_specs=[_row_spec(din), _full_spec(2, din), _full_spec(1, din),
                  _full_spec(1, din), _full_spec(din, dout),
                  _full_spec(1, dout)],
        out_specs=[_row_spec(dout), _full_spec(2, dout)],
        out_shape=[jax.ShapeDtypeStruct((B, dout), jnp.bfloat16),
                   jax.ShapeDtypeStruct((2, dout), jnp.float32)],
        compiler_params=_ARB,
    )(h, st, g, be, W, b)


def _final(h, st, g, be, W, b, din, dout):
    return pl.pallas_call(
        _fin_body,
        grid=(GRID,),
        in_specs=[_row_spec(din), _full_spec(2, din), _full_spec(1, din),
                  _full_spec(1, din), _full_spec(din, dout),
                  _full_spec(1, dout)],
        out_specs=_row_spec(dout),
        out_shape=jax.ShapeDtypeStruct((B, dout), jnp.float32),
        compiler_params=_ARB,
    )(h, st, g, be, W, b)


def kernel(seq, pos, side, champ_w, pos_w, side_w,
           W1, b1, g1, be1, W2, b2, g2, be2, W3, b3, g3, be3,
           Wout, bout):
    h2d = _sc_embed(seq.astype(jnp.int32).reshape(-1),
                    jnp.zeros((_CH, HP), jnp.float32))

    cw = jnp.pad(champ_w, ((0, HP - VOCAB), (0, 0))).astype(jnp.bfloat16)
    pw = jnp.pad(pos_w, ((0, EMB - pos_w.shape[0]), (0, 0))).astype(jnp.bfloat16)
    sw = jnp.pad(side_w, ((0, EMB - side_w.shape[0]), (0, 0))).astype(jnp.bfloat16)
    pos_f = pos.astype(jnp.int32).reshape(B, 1)
    side_f = side.astype(jnp.int32).reshape(B, 1)
    W1b = W1.astype(jnp.bfloat16)
    W2b = W2.astype(jnp.bfloat16)
    W3b = W3.astype(jnp.bfloat16)
    Wob = Wout.astype(jnp.bfloat16)
    bo = bout.reshape(1, VOCAB)

    h1, st1 = _layer1(h2d, pos_f, side_f, cw, pw, sw, W1b, b1.reshape(1, -1))
    h2, st2 = _mid(h1, st1, g1.reshape(1, -1), be1.reshape(1, -1),
                   W2b, b2.reshape(1, -1), 1024, 1024)
    h3, st3 = _mid(h2, st2, g2.reshape(1, -1), be2.reshape(1, -1),
                   W3b, b3.reshape(1, -1), 1024, 512)
    return _final(h3, st3, g3.reshape(1, -1), be3.reshape(1, -1),
                  Wob, bo, 512, VOCAB)


# speedup vs baseline: 1.0543x; 1.0543x over previous
import functools

import jax
import jax.numpy as jnp
from jax import lax
from jax.experimental import pallas as pl
from jax.experimental.pallas import tpu as pltpu
from jax.experimental.pallas import tpu_sc as plsc

B = 4096
EMB = 128
VOCAB = 1000
HP = 1024
L = 50
EPS = 1e-5

_NC, _NS = 2, 16
_NW = _NC * _NS
_RPW = B // _NW
_CH = 32
_NCH = _RPW // _CH


def _sc_body(seq_hbm, zz_hbm, h_hbm,
             seqb0, seqb1, hb0, hb1, sem0, sem1):
    wid = lax.axis_index("s") * _NC + lax.axis_index("c")
    base = wid * _RPW

    zeros16 = jnp.zeros((16,), jnp.float32)
    ones16 = jnp.ones((16,), jnp.float32)
    tail = lax.iota(jnp.int32, 16) >= jnp.int32(4 * 16 - L)

    cz0 = pltpu.async_copy(zz_hbm, hb0, sem0)
    cz1 = pltpu.async_copy(zz_hbm, hb1, sem1)
    cz0.wait()
    cz1.wait()

    _UNR = 4

    def _rows(seqb, hb):
        def body(i, carry):
            for di in range(_UNR):
                r = i * _UNR + di
                row = jnp.broadcast_to(r, (16,)).astype(jnp.int32)
                for j in range(3):
                    col = seqb[pl.ds(r * L + j * 16, 16)]
                    plsc.addupdate_scatter(hb, [row, col], ones16)
                col = seqb[pl.ds(r * L + (L - 16), 16)]
                plsc.addupdate_scatter(hb, [row, col], ones16, mask=tail)
            return carry
        lax.fori_loop(0, _CH // _UNR, body, 0)

    def _unrows(seqb, hb):
        def body(i, carry):
            for di in range(_UNR):
                r = i * _UNR + di
                row = jnp.broadcast_to(r, (16,)).astype(jnp.int32)
                for j in range(3):
                    col = seqb[pl.ds(r * L + j * 16, 16)]
                    plsc.store_scatter(hb, [row, col], zeros16)
                col = seqb[pl.ds(r * L + (L - 16), 16)]
                plsc.store_scatter(hb, [row, col], zeros16, mask=tail)
            return carry
        lax.fori_loop(0, _CH // _UNR, body, 0)

    bufs = ((seqb0, hb0, sem0), (seqb1, hb1, sem1))
    cps = [None, None]
    for c in range(_NCH):
        seqb, hb, sem = bufs[c % 2]
        if cps[c % 2] is not None:
            cps[c % 2].wait()
            _unrows(seqb, hb)
        r0 = base + c * _CH
        pltpu.sync_copy(seq_hbm.at[pl.ds(r0 * L, _CH * L)], seqb)
        _rows(seqb, hb)
        cps[c % 2] = pltpu.async_copy(hb, h_hbm.at[pl.ds(r0, _CH)], sem)
    cps[0].wait()
    cps[1].wait()


@functools.cache
def _sc_embed_call():
    return pl.kernel(
        _sc_body,
        out_type=jax.ShapeDtypeStruct((B, HP), jnp.float32),
        mesh=plsc.VectorSubcoreMesh(core_axis_name="c", subcore_axis_name="s",
                                    num_cores=_NC, num_subcores=_NS),
        scratch_types=[
            pltpu.VMEM((_CH * L,), jnp.int32),
            pltpu.VMEM((_CH * L,), jnp.int32),
            pltpu.VMEM((_CH, HP), jnp.float32),
            pltpu.VMEM((_CH, HP), jnp.float32),
            pltpu.SemaphoreType.DMA,
            pltpu.SemaphoreType.DMA,
        ],
        compiler_params=pltpu.CompilerParams(needs_layout_passes=False),
    )


def _sc_embed(*args):
    return _sc_embed_call()(*args)

TILE = 1024
GRID = B // TILE


def _bn_fold(st_s, g_ref, be_ref):
    mu = st_s[0:1, :] * (1.0 / B)
    var = st_s[1:2, :] * (1.0 / B) - mu * mu
    a = g_ref[...] * lax.rsqrt(var + EPS)
    c = be_ref[...] - mu * a
    return a, c


def _stats(st_s, i, h32):
    @pl.when(i == 0)
    def _():
        st_s[...] = jnp.zeros_like(st_s)
    st_s[...] += jnp.concatenate(
        [jnp.sum(h32, 0, keepdims=True),
         jnp.sum(h32 * h32, 0, keepdims=True)], 0)


def _fused_body(h_ref, pos_ref, side_ref, cw_ref, pw_ref, sw_ref,
                W1_ref, b1_ref, g1_ref, be1_ref,
                W2_ref, b2_ref, g2_ref, be2_ref,
                W3_ref, b3_ref, g3_ref, be3_ref,
                Wo_ref, bo_ref, o_ref,
                h1_s, h2_s, h3_s, st1_s, st2_s, st3_s):
    s = pl.program_id(0)
    i = pl.program_id(1)
    r = pl.ds(i * TILE, TILE)

    @pl.when(s == 0)
    def _l1():
        hb = h_ref[...].astype(jnp.bfloat16)
        sv = jnp.dot(hb, cw_ref[...], preferred_element_type=jnp.float32)
        n = jnp.maximum(jnp.float32(L) - h_ref[:, 0:1], 1.0)
        m = sv / n
        lanes = lax.broadcasted_iota(jnp.int32, (TILE, EMB), 1)
        oh_p = (pos_ref[...] == lanes).astype(jnp.bfloat16)
        oh_s = (side_ref[...] == lanes).astype(jnp.bfloat16)
        p = jnp.dot(oh_p, pw_ref[...], preferred_element_type=jnp.float32)
        d = jnp.dot(oh_s, sw_ref[...], preferred_element_type=jnp.float32)
        x = jnp.concatenate([sv, m, p, d], axis=1)
        y = jnp.dot(x.astype(jnp.bfloat16), W1_ref[...],
                    preferred_element_type=jnp.float32) + b1_ref[...]
        hb1 = jnp.maximum(y, 0.0).astype(jnp.bfloat16)
        h1_s[r, :] = hb1
        _stats(st1_s, i, hb1.astype(jnp.float32))

    @pl.when(s == 1)
    def _l2():
        a, c = _bn_fold(st1_s, g1_ref, be1_ref)
        z = (h1_s[r, :].astype(jnp.float32) * a + c).astype(jnp.bfloat16)
        y = jnp.dot(z, W2_ref[...],
                    preferred_element_type=jnp.float32) + b2_ref[...]
        hb = jnp.maximum(y, 0.0).astype(jnp.bfloat16)
        h2_s[r, :] = hb
        _stats(st2_s, i, hb.astype(jnp.float32))

    @pl.when(s == 2)
    def _l3():
        a, c = _bn_fold(st2_s, g2_ref, be2_ref)
        z = (h2_s[r, :].astype(jnp.float32) * a + c).astype(jnp.bfloat16)
        y = jnp.dot(z, W3_ref[...],
                    preferred_element_type=jnp.float32) + b3_ref[...]
        hb = jnp.maximum(y, 0.0).astype(jnp.bfloat16)
        h3_s[r, :] = hb
        _stats(st3_s, i, hb.astype(jnp.float32))

    @pl.when(s == 3)
    def _l4():
        a, c = _bn_fold(st3_s, g3_ref, be3_ref)
        z = (h3_s[r, :].astype(jnp.float32) * a + c).astype(jnp.bfloat16)
        o_ref[...] = jnp.dot(z, Wo_ref[...],
                             preferred_element_type=jnp.float32) + bo_ref[...]


def _tc_mlp(h2d, pos_f, side_f, cw, pw, sw, W1b, b1r, g1r, be1r,
            W2b, b2r, g2r, be2r, W3b, b3r, g3r, be3r, Wob, bor):
    def row0(s, i):
        return (jnp.where(s == 0, i, 0), 0)

    def row3(s, i):
        return (jnp.where(s == 3, i, 0), 0)

    def full(s, i):
        return (0, 0)

    def fs(m, n):
        return pl.BlockSpec((m, n), full)

    return pl.pallas_call(
        _fused_body,
        grid=(4, GRID),
        in_specs=[
            pl.BlockSpec((TILE, HP), row0),
            pl.BlockSpec((TILE, 1), row0),
            pl.BlockSpec((TILE, 1), row0),
            fs(HP, EMB), fs(EMB, EMB), fs(EMB, EMB),
            fs(4 * EMB, 1024), fs(1, 1024), fs(1, 1024), fs(1, 1024),
            fs(1024, 1024), fs(1, 1024), fs(1, 1024), fs(1, 1024),
            fs(1024, 512), fs(1, 512), fs(1, 512), fs(1, 512),
            fs(512, VOCAB), fs(1, VOCAB),
        ],
        out_specs=pl.BlockSpec((TILE, VOCAB), row3),
        out_shape=jax.ShapeDtypeStruct((B, VOCAB), jnp.float32),
        scratch_shapes=[
            pltpu.VMEM((B, 1024), jnp.bfloat16),
            pltpu.VMEM((B, 1024), jnp.bfloat16),
            pltpu.VMEM((B, 512), jnp.bfloat16),
            pltpu.VMEM((2, 1024), jnp.float32),
            pltpu.VMEM((2, 1024), jnp.float32),
            pltpu.VMEM((2, 512), jnp.float32),
        ],
        compiler_params=pltpu.CompilerParams(
            dimension_semantics=("arbitrary", "arbitrary"),
            vmem_limit_bytes=100 * 1024 * 1024,
        ),
    )(h2d, pos_f, side_f, cw, pw, sw, W1b, b1r, g1r, be1r,
      W2b, b2r, g2r, be2r, W3b, b3r, g3r, be3r, Wob, bor)


def kernel(seq, pos, side, champ_w, pos_w, side_w,
           W1, b1, g1, be1, W2, b2, g2, be2, W3, b3, g3, be3,
           Wout, bout):
    h2d = _sc_embed(seq.astype(jnp.int32).reshape(-1),
                    jnp.zeros((_CH, HP), jnp.float32))

    cw = jnp.pad(champ_w, ((0, HP - VOCAB), (0, 0))).astype(jnp.bfloat16)
    pw = jnp.pad(pos_w, ((0, EMB - pos_w.shape[0]), (0, 0))).astype(jnp.bfloat16)
    sw = jnp.pad(side_w, ((0, EMB - side_w.shape[0]), (0, 0))).astype(jnp.bfloat16)
    pos_f = pos.astype(jnp.int32).reshape(B, 1)
    side_f = side.astype(jnp.int32).reshape(B, 1)
    W1b = W1.astype(jnp.bfloat16)
    W2b = W2.astype(jnp.bfloat16)
    W3b = W3.astype(jnp.bfloat16)
    Wob = Wout.astype(jnp.bfloat16)
    bo = bout.reshape(1, VOCAB)

    return _tc_mlp(h2d, pos_f, side_f, cw, pw, sw,
                   W1b, b1.reshape(1, -1), g1.reshape(1, -1),
                   be1.reshape(1, -1),
                   W2b, b2.reshape(1, -1), g2.reshape(1, -1),
                   be2.reshape(1, -1),
                   W3b, b3.reshape(1, -1), g3.reshape(1, -1),
                   be3.reshape(1, -1), Wob, bo)
